# Initial kernel scaffold; baseline (speedup 1.0000x reference)
import jax
import jax.numpy as jnp
from jax.experimental import pallas as pl


def kernel(x, edge_index, edge_attr, batch, W_in, b_in, W_e0, b_e0, W_e1, b_e1, W_msg0, b_msg0, W_up0, b_up0, W_msg1, b_msg1, W_up1, b_up1, W_on, b_on, W_og, b_og, W_cf, b_cf, W_gcu, b_gcu, W_nphi, b_nphi):
    raise NotImplementedError("write your pallas kernel here")


# speedup vs baseline: 3.4761x; 3.4761x over previous
import functools

import jax
import jax.numpy as jnp
from jax import lax
from jax.experimental import pallas as pl
from jax.experimental.pallas import tpu as pltpu
from jax.experimental.pallas import tpu_sc as plsc

_NC = 2
_NS = 16
_LANES = 16

_DH = 128
_G = 16


def _sc_message_pass(N, E):
  EPW = E // (_NC * _NS)
  K = 128
  NFULL = EPW // K
  REM = EPW - NFULL * K
  RPB = 8 * (N // (8 * _NS))
  REXTRA = N - RPB * _NS
  assert EPW * _NC * _NS == E and N % 8 == 0
  assert EPW % 8 == 0 and (REM == 0 or REM % 8 == 0)
  assert REXTRA % 8 == 0 and REXTRA < K

  mesh = plsc.VectorSubcoreMesh(core_axis_name="c", subcore_axis_name="s")
  scratch = [
      pltpu.VMEM_SHARED((N, _DH), jnp.float32),
      pltpu.VMEM((K,), jnp.int32),
      pltpu.VMEM((K,), jnp.int32),
      pltpu.VMEM((K, _DH), jnp.float32),
      pltpu.VMEM((K, _DH), jnp.float32),
      pltpu.SemaphoreType.DMA,
      pltpu.SemaphoreType.DMA,
  ]
  if REM:
    scratch += [pltpu.VMEM((REM,), jnp.int32), pltpu.VMEM((REM,), jnp.int32)]

  @functools.partial(
      pl.kernel,
      out_type=jax.ShapeDtypeStruct((_NC * N, _DH), jnp.float32),
      mesh=mesh,
      scratch_types=scratch,
  )
  def body(hm, em, srci, dsti, out, agg_sh, isrc, idst, embuf, rowbuf,
           gsem, esem, *rem_bufs):
    cid = lax.axis_index("c")
    sid = lax.axis_index("s")

    def zrow(r, _):
      for c in range(_DH // _LANES):
        rowbuf[r, pl.ds(c * _LANES, _LANES)] = jnp.zeros((_LANES,), jnp.float32)
      return 0
    lax.fori_loop(0, K, zrow, 0)
    zbase = pl.multiple_of(sid * RPB, 8)
    nz = RPB // K
    rz = RPB - nz * K
    for b in range(nz):
      pltpu.sync_copy(rowbuf, agg_sh.at[pl.ds(zbase + b * K, K)])
    if rz:
      pltpu.sync_copy(rowbuf.at[pl.ds(0, rz)],
                      agg_sh.at[pl.ds(zbase + nz * K, rz)])
    if REXTRA:
      @pl.when(sid == _NS - 1)
      def _():
        pltpu.sync_copy(rowbuf.at[pl.ds(0, REXTRA)],
                        agg_sh.at[pl.ds(N - REXTRA, REXTRA)])
    plsc.subcore_barrier()

    base0 = (cid * _NS + sid) * EPW

    def chunk(base, k, isrc_b, idst_b):
      base = pl.multiple_of(base, 8)
      pltpu.sync_copy(srci.at[pl.ds(base, k)], isrc_b)
      pltpu.sync_copy(dsti.at[pl.ds(base, k)], idst_b)
      g = pltpu.async_copy(hm.at[isrc_b], rowbuf.at[pl.ds(0, k)], gsem)
      e = pltpu.async_copy(em.at[pl.ds(base, k)], embuf.at[pl.ds(0, k)], esem)
      g.wait()
      e.wait()

      def rbody(r, _):
        for c in range(_DH // _LANES):
          sl = pl.ds(c * _LANES, _LANES)
          embuf[r, sl] = jnp.maximum(rowbuf[r, sl] + embuf[r, sl], 0.0)
        return 0
      lax.fori_loop(0, k, rbody, 0)
      pltpu.sync_copy(embuf.at[pl.ds(0, k)], agg_sh.at[idst_b], add=True)

    def cbody(ci, _):
      chunk(base0 + ci * K, K, isrc, idst)
      return 0
    lax.fori_loop(0, NFULL, cbody, 0)
    if REM:
      chunk(base0 + NFULL * K, REM, rem_bufs[0], rem_bufs[1])

    plsc.subcore_barrier()
    obase = pl.multiple_of(cid * N + sid * RPB, 8)
    pltpu.sync_copy(agg_sh.at[pl.ds(zbase, RPB)], out.at[pl.ds(obase, RPB)])
    if REXTRA:
      @pl.when(sid == _NS - 1)
      def _():
        xbase = pl.multiple_of(cid * N + N - REXTRA, 8)
        pltpu.sync_copy(agg_sh.at[pl.ds(N - REXTRA, REXTRA)],
                        out.at[pl.ds(xbase, REXTRA)])

  return body


def _dot(a, b):
  return jnp.dot(a, b, preferred_element_type=jnp.float32)


def _em_body(ea, We0, be0, Wm0, bm0, We1, be1, Wm1, bm1, em0, em1):
  a = ea[...]
  Wm0e = Wm0[_DH:, :]
  em0[...] = _dot(a, _dot(We0[...], Wm0e)) + (_dot(be0[...], Wm0e) + bm0[...])
  Wm1e = Wm1[_DH:, :]
  em1[...] = _dot(a, _dot(We1[...], Wm1e)) + (_dot(be1[...], Wm1e) + bm1[...])


def _pre_body(x, Win, bin_, Wm0, h_out, hm_out):
  h = _dot(x[...], Win[...]) + bin_[...]
  h_out[...] = h
  hm_out[...] = _dot(h, Wm0[: _DH, :])


def _upd_body(h, a, Wu, bu, Wm, ho, hm):
  agg = a[0] + a[1]
  hn = _dot(h[...], Wu[: _DH, :]) + _dot(agg, Wu[_DH:, :]) + bu[...]
  hn = jnp.maximum(hn, 0.0)
  ho[...] = hn
  hm[...] = _dot(hn, Wm[: _DH, :])


def _upd_last_body(h, a, Wu, bu, ho):
  agg = a[0] + a[1]
  hn = _dot(h[...], Wu[: _DH, :]) + _dot(agg, Wu[_DH:, :]) + bu[...]
  ho[...] = jnp.maximum(hn, 0.0)


def _readout_body(h, b, Wog, bog, Wcf, bcf, Wgcu, bgcu, Wnphi, ctx2,
                  pooled, cnt):
  i = pl.program_id(0)

  @pl.when(i == 0)
  def _():
    pooled[...] = jnp.zeros_like(pooled)
    cnt[...] = jnp.zeros_like(cnt)

  bn = h.shape[0]
  oh = (b[...] == lax.broadcasted_iota(jnp.int32, (bn, _G), 1))
  oh = oh.astype(jnp.float32)
  dn = (((0,), (0,)), ((), ()))
  pooled[...] += lax.dot_general(oh, h[...], dn,
                                 preferred_element_type=jnp.float32)
  cnt[...] += lax.dot_general(oh, jnp.ones((bn, _DH), jnp.float32), dn,
                              preferred_element_type=jnp.float32)

  @pl.when(i == pl.num_programs(0) - 1)
  def _():
    pm = pooled[...] / jnp.maximum(cnt[...], 1.0)
    gr = _dot(pm, Wog[...]) + bog[...]
    cx = jnp.maximum(_dot(gr, Wcf[...]) + bcf[...], 0.0)
    cx = _dot(cx, Wgcu[...]) + bgcu[...]
    ctx2[...] = _dot(cx, Wnphi[_DH:, :])


def _fin_body(h, b, ctx2, Won, bon, Wnphi, bnphi, out):
  Wn1 = Wnphi[: _DH, :]
  Wno = _dot(Won[...], Wn1)
  bno = _dot(bon[...], Wn1) + bnphi[...]
  bn = h.shape[0]
  oh = (b[...] == lax.broadcasted_iota(jnp.int32, (bn, _G), 1))
  oh = oh.astype(jnp.float32)
  out[...] = _dot(h[...], Wno) + bno + _dot(oh, ctx2[...])


def _full(shape):
  return pl.BlockSpec(shape, lambda i: tuple(0 for _ in shape))


def kernel(x, edge_index, edge_attr, batch,
           W_in, b_in, W_e0, b_e0, W_e1, b_e1,
           W_msg0, b_msg0, W_up0, b_up0,
           W_msg1, b_msg1, W_up1, b_up1,
           W_on, b_on, W_og, b_og,
           W_cf, b_cf, W_gcu, b_gcu,
           W_nphi, b_nphi):
  N, DF = x.shape
  E = edge_attr.shape[0]
  DE = edge_attr.shape[1]
  DEE = W_e0.shape[1]
  f32 = jnp.float32

  src = edge_index[0]
  dst = edge_index[1]
  batch2 = batch.reshape(N, 1)
  b_in2 = b_in.reshape(1, -1)
  b_e02 = b_e0.reshape(1, -1)
  b_e12 = b_e1.reshape(1, -1)
  b_msg02 = b_msg0.reshape(1, -1)
  b_msg12 = b_msg1.reshape(1, -1)
  b_up02 = b_up0.reshape(1, -1)
  b_up12 = b_up1.reshape(1, -1)
  b_on2 = b_on.reshape(1, -1)
  b_og2 = b_og.reshape(1, -1)
  b_cf2 = b_cf.reshape(1, -1)
  b_gcu2 = b_gcu.reshape(1, -1)
  b_nphi2 = b_nphi.reshape(1, -1)

  BN = 1000
  BE = 4000
  ngrid = N // BN

  em0, em1 = pl.pallas_call(
      _em_body,
      grid=(E // BE,),
      in_specs=[
          pl.BlockSpec((BE, DE), lambda i: (i, 0)),
          _full((DE, DEE)), _full((1, DEE)), _full((DEE + _DH, _DH)),
          _full((1, _DH)),
          _full((DE, DEE)), _full((1, DEE)), _full((DEE + _DH, _DH)),
          _full((1, _DH)),
      ],
      out_specs=[pl.BlockSpec((BE, _DH), lambda i: (i, 0))] * 2,
      out_shape=[jax.ShapeDtypeStruct((E, _DH), f32)] * 2,
  )(edge_attr, W_e0, b_e02, W_msg0, b_msg02, W_e1, b_e12, W_msg1, b_msg12)

  h0, hm0 = pl.pallas_call(
      _pre_body,
      grid=(ngrid,),
      in_specs=[
          pl.BlockSpec((BN, DF), lambda i: (i, 0)),
          _full((DF, _DH)), _full((1, _DH)), _full((DEE + _DH, _DH)),
      ],
      out_specs=[pl.BlockSpec((BN, _DH), lambda i: (i, 0))] * 2,
      out_shape=[jax.ShapeDtypeStruct((N, _DH), f32)] * 2,
  )(x, W_in, b_in2, W_msg0)

  sc_pass = _sc_message_pass(N, E)
  aggp0 = sc_pass(hm0, em0, src, dst).reshape(_NC, N, _DH)

  h1, hm1 = pl.pallas_call(
      _upd_body,
      grid=(ngrid,),
      in_specs=[
          pl.BlockSpec((BN, _DH), lambda i: (i, 0)),
          pl.BlockSpec((_NC, BN, _DH), lambda i: (0, i, 0)),
          _full((2 * _DH, _DH)), _full((1, _DH)), _full((DEE + _DH, _DH)),
      ],
      out_specs=[pl.BlockSpec((BN, _DH), lambda i: (i, 0))] * 2,
      out_shape=[jax.ShapeDtypeStruct((N, _DH), f32)] * 2,
  )(h0, aggp0, W_up0, b_up02, W_msg1)

  aggp1 = sc_pass(hm1, em1, src, dst).reshape(_NC, N, _DH)

  h2 = pl.pallas_call(
      _upd_last_body,
      grid=(ngrid,),
      in_specs=[
          pl.BlockSpec((BN, _DH), lambda i: (i, 0)),
          pl.BlockSpec((_NC, BN, _DH), lambda i: (0, i, 0)),
          _full((2 * _DH, _DH)), _full((1, _DH)),
      ],
      out_specs=pl.BlockSpec((BN, _DH), lambda i: (i, 0)),
      out_shape=jax.ShapeDtypeStruct((N, _DH), f32),
  )(h1, aggp1, W_up1, b_up12)

  ctx2 = pl.pallas_call(
      _readout_body,
      grid=(ngrid,),
      in_specs=[
          pl.BlockSpec((BN, _DH), lambda i: (i, 0)),
          pl.BlockSpec((BN, 1), lambda i: (i, 0)),
          _full((_DH, _DH)), _full((1, _DH)),
          _full((_DH, _DH)), _full((1, _DH)),
          _full((_DH, _DH)), _full((1, _DH)),
          _full((2 * _DH, 2)),
      ],
      out_specs=pl.BlockSpec((_G, 2), lambda i: (0, 0)),
      out_shape=jax.ShapeDtypeStruct((_G, 2), f32),
      scratch_shapes=[
          pltpu.VMEM((_G, _DH), f32),
          pltpu.VMEM((_G, _DH), f32),
      ],
  )(h2, batch2, W_og, b_og2, W_cf, b_cf2, W_gcu, b_gcu2, W_nphi)

  node_out = pl.pallas_call(
      _fin_body,
      grid=(ngrid,),
      in_specs=[
          pl.BlockSpec((BN, _DH), lambda i: (i, 0)),
          pl.BlockSpec((BN, 1), lambda i: (i, 0)),
          _full((_G, 2)),
          _full((_DH, _DH)), _full((1, _DH)),
          _full((2 * _DH, 2)), _full((1, 2)),
      ],
      out_specs=pl.BlockSpec((BN, 2), lambda i: (i, 0)),
      out_shape=jax.ShapeDtypeStruct((N, 2), f32),
  )(h2, batch2, ctx2, W_on, b_on2, W_nphi, b_nphi2)

  return node_out


# speedup vs baseline: 5.0571x; 1.4548x over previous
import functools

import jax
import jax.numpy as jnp
from jax import lax
from jax.experimental import pallas as pl
from jax.experimental.pallas import tpu as pltpu
from jax.experimental.pallas import tpu_sc as plsc

_NC = 2
_NS = 16
_LANES = 16

_DH = 128
_G = 16


def _sc_message_pass(N, E):
  EPW = E // (_NC * _NS)
  K = 64
  NFULL = EPW // K
  REM = EPW - NFULL * K
  RPB = 8 * (N // (8 * _NS))
  REXTRA = N - RPB * _NS
  assert EPW * _NC * _NS == E and N % 8 == 0
  assert EPW % 8 == 0 and (REM == 0 or REM % 8 == 0)
  assert REXTRA % 8 == 0 and REXTRA < K

  ND = 2
  NI = 3
  UNROLL = 6
  assert NFULL % UNROLL == 0
  NSUP = NFULL // UNROLL

  mesh = plsc.VectorSubcoreMesh(core_axis_name="c", subcore_axis_name="s")
  scratch = (
      [pltpu.VMEM_SHARED((N, _DH), jnp.float32)]
      + [pltpu.VMEM((K,), jnp.int32)] * (2 * NI)
      + [pltpu.VMEM((K, _DH), jnp.float32)] * (2 * ND)
      + [pltpu.SemaphoreType.DMA] * (NI + 2 * ND)
      + ([pltpu.VMEM((REM,), jnp.int32)] * 2 if REM else [])
  )

  @functools.partial(
      pl.kernel,
      out_type=jax.ShapeDtypeStruct((_NC * N, _DH), jnp.float32),
      mesh=mesh,
      scratch_types=scratch,
  )
  def body(hm, em, srci, dsti, out, agg_sh, *bufs):
    isrc = bufs[0:NI]
    idst = bufs[NI:2 * NI]
    embuf = bufs[2 * NI:2 * NI + ND]
    rowbuf = bufs[2 * NI + ND:2 * NI + 2 * ND]
    isem = bufs[2 * NI + 2 * ND:2 * NI + 2 * ND + NI]
    gsem = bufs[2 * NI + 2 * ND + NI:2 * NI + 2 * ND + NI + ND]
    esem = bufs[2 * NI + 2 * ND + NI + ND:2 * NI + 2 * ND + NI + 2 * ND]
    rem_bufs = bufs[2 * NI + 2 * ND + NI + 2 * ND:]

    cid = lax.axis_index("c")
    sid = lax.axis_index("s")

    zb = rowbuf[0]
    def zrow(r, _):
      for c in range(_DH // _LANES):
        zb[r, pl.ds(c * _LANES, _LANES)] = jnp.zeros((_LANES,), jnp.float32)
      return 0
    lax.fori_loop(0, K, zrow, 0)
    zbase = pl.multiple_of(sid * RPB, 8)
    nz = RPB // K
    rz = RPB - nz * K
    for b in range(nz):
      pltpu.sync_copy(zb, agg_sh.at[pl.ds(zbase + b * K, K)])
    if rz:
      pltpu.sync_copy(zb.at[pl.ds(0, rz)],
                      agg_sh.at[pl.ds(zbase + nz * K, rz)])
    if REXTRA:
      @pl.when(sid == _NS - 1)
      def _():
        pltpu.sync_copy(zb.at[pl.ds(0, REXTRA)],
                        agg_sh.at[pl.ds(N - REXTRA, REXTRA)])
    plsc.subcore_barrier()

    base0 = (cid * _NS + sid) * EPW

    def ebase(c):
      return pl.multiple_of(base0 + c * K, 8)

    def issue_idx(c, i):
      pltpu.async_copy(srci.at[pl.ds(ebase(c), K)], isrc[i], isem[i])
      pltpu.async_copy(dsti.at[pl.ds(ebase(c), K)], idst[i], isem[i])

    def wait_idx(i):
      pltpu.make_async_copy(srci.at[pl.ds(0, K)], isrc[i], isem[i]).wait()
      pltpu.make_async_copy(dsti.at[pl.ds(0, K)], idst[i], isem[i]).wait()

    def issue_data(c, i, d):
      pltpu.async_copy(hm.at[isrc[i]], rowbuf[d], gsem[d])
      pltpu.async_copy(em.at[pl.ds(ebase(c), K)], embuf[d], esem[d])

    def wait_data(d):
      pltpu.make_async_copy(hm.at[pl.ds(0, K)], rowbuf[d], gsem[d]).wait()
      pltpu.make_async_copy(em.at[pl.ds(0, K)], embuf[d], esem[d]).wait()

    def compute_scatter(d, i):
      emb = embuf[d]
      rwb = rowbuf[d]

      @plsc.parallel_loop(0, K, 1, unroll=2)
      def _(r):
        for c in range(_DH // _LANES):
          sl = pl.ds(c * _LANES, _LANES)
          emb[r, sl] = jnp.maximum(rwb[r, sl] + emb[r, sl], 0.0)

      pltpu.sync_copy(emb, agg_sh.at[idst[i]], add=True)

    issue_idx(0, 0)
    issue_idx(1, 1)
    wait_idx(0)
    issue_data(0, 0, 0)

    def super_body(it, _):
      for j in range(UNROLL):
        c = it * UNROLL + j
        nxt_i = (j + 1) % NI
        nxt_d = (j + 1) % ND
        if j + 1 < UNROLL:
          wait_idx(nxt_i)
          issue_data(c + 1, nxt_i, nxt_d)
        else:
          @pl.when(it < NSUP - 1)
          def _(nxt_i=nxt_i, nxt_d=nxt_d, c=c):
            wait_idx(nxt_i)
            issue_data(c + 1, nxt_i, nxt_d)
        if j + 2 < UNROLL:
          issue_idx(c + 2, (j + 2) % NI)
        else:
          @pl.when(it < NSUP - 1)
          def _(c=c, j=j):
            issue_idx(c + 2, (j + 2) % NI)
        wait_data(j % ND)
        compute_scatter(j % ND, j % NI)
      return 0

    lax.fori_loop(0, NSUP, super_body, 0)

    if REM:
      isr, idr = rem_bufs
      baser = pl.multiple_of(base0 + NFULL * K, 8)
      pltpu.sync_copy(srci.at[pl.ds(baser, REM)], isr)
      pltpu.sync_copy(dsti.at[pl.ds(baser, REM)], idr)
      g = pltpu.async_copy(hm.at[isr], rowbuf[0].at[pl.ds(0, REM)], gsem[0])
      e = pltpu.async_copy(em.at[pl.ds(baser, REM)], embuf[0].at[pl.ds(0, REM)],
                           esem[0])
      g.wait()
      e.wait()
      emb = embuf[0]
      rwb = rowbuf[0]

      @plsc.parallel_loop(0, REM, 1, unroll=2)
      def _(r):
        for cc in range(_DH // _LANES):
          sl = pl.ds(cc * _LANES, _LANES)
          emb[r, sl] = jnp.maximum(rwb[r, sl] + emb[r, sl], 0.0)

      pltpu.sync_copy(emb.at[pl.ds(0, REM)], agg_sh.at[idr], add=True)

    plsc.subcore_barrier()
    obase = pl.multiple_of(cid * N + sid * RPB, 8)
    pltpu.sync_copy(agg_sh.at[pl.ds(zbase, RPB)], out.at[pl.ds(obase, RPB)])
    if REXTRA:
      @pl.when(sid == _NS - 1)
      def _():
        xbase = pl.multiple_of(cid * N + N - REXTRA, 8)
        pltpu.sync_copy(agg_sh.at[pl.ds(N - REXTRA, REXTRA)],
                        out.at[pl.ds(xbase, REXTRA)])

  return body


def _dot(a, b):
  return jnp.dot(a, b, preferred_element_type=jnp.float32)


def _em_body(ea, We, be, Wm, bm, em):
  Wme = Wm[_DH:, :]
  em[...] = _dot(ea[...], _dot(We[...], Wme)) + (_dot(be[...], Wme) + bm[...])


def _pre_body(x, Win, bin_, Wm0, h_out, hm_out):
  h = _dot(x[...], Win[...]) + bin_[...]
  h_out[...] = h
  hm_out[...] = _dot(h, Wm0[: _DH, :])


def _upd_body(h, a, Wu, bu, Wm, ho, hm):
  agg = a[0] + a[1]
  hn = _dot(h[...], Wu[: _DH, :]) + _dot(agg, Wu[_DH:, :]) + bu[...]
  hn = jnp.maximum(hn, 0.0)
  ho[...] = hn
  hm[...] = _dot(hn, Wm[: _DH, :])


def _upd_last_body(h, a, Wu, bu, ho):
  agg = a[0] + a[1]
  hn = _dot(h[...], Wu[: _DH, :]) + _dot(agg, Wu[_DH:, :]) + bu[...]
  ho[...] = jnp.maximum(hn, 0.0)


def _readout_body(h, b, Wog, bog, Wcf, bcf, Wgcu, bgcu, Wnphi, ctx2,
                  pooled, cnt):
  i = pl.program_id(0)

  @pl.when(i == 0)
  def _():
    pooled[...] = jnp.zeros_like(pooled)
    cnt[...] = jnp.zeros_like(cnt)

  bn = h.shape[0]
  oh = (b[...] == lax.broadcasted_iota(jnp.int32, (bn, _G), 1))
  oh = oh.astype(jnp.float32)
  dn = (((0,), (0,)), ((), ()))
  pooled[...] += lax.dot_general(oh, h[...], dn,
                                 preferred_element_type=jnp.float32)
  cnt[...] += lax.dot_general(oh, jnp.ones((bn, _DH), jnp.float32), dn,
                              preferred_element_type=jnp.float32)

  @pl.when(i == pl.num_programs(0) - 1)
  def _():
    pm = pooled[...] / jnp.maximum(cnt[...], 1.0)
    gr = _dot(pm, Wog[...]) + bog[...]
    cx = jnp.maximum(_dot(gr, Wcf[...]) + bcf[...], 0.0)
    cx = _dot(cx, Wgcu[...]) + bgcu[...]
    ctx2[...] = _dot(cx, Wnphi[_DH:, :])


def _fin_body(h, b, ctx2, Won, bon, Wnphi, bnphi, out):
  Wn1 = Wnphi[: _DH, :]
  Wno = _dot(Won[...], Wn1)
  bno = _dot(bon[...], Wn1) + bnphi[...]
  bn = h.shape[0]
  oh = (b[...] == lax.broadcasted_iota(jnp.int32, (bn, _G), 1))
  oh = oh.astype(jnp.float32)
  out[...] = _dot(h[...], Wno) + bno + _dot(oh, ctx2[...])


def _full(shape):
  return pl.BlockSpec(shape, lambda i: tuple(0 for _ in shape))


def kernel(x, edge_index, edge_attr, batch,
           W_in, b_in, W_e0, b_e0, W_e1, b_e1,
           W_msg0, b_msg0, W_up0, b_up0,
           W_msg1, b_msg1, W_up1, b_up1,
           W_on, b_on, W_og, b_og,
           W_cf, b_cf, W_gcu, b_gcu,
           W_nphi, b_nphi):
  N, DF = x.shape
  E = edge_attr.shape[0]
  DE = edge_attr.shape[1]
  DEE = W_e0.shape[1]
  f32 = jnp.float32

  src = edge_index[0]
  dst = edge_index[1]
  batch2 = batch.reshape(N, 1)
  b_in2 = b_in.reshape(1, -1)
  b_e02 = b_e0.reshape(1, -1)
  b_e12 = b_e1.reshape(1, -1)
  b_msg02 = b_msg0.reshape(1, -1)
  b_msg12 = b_msg1.reshape(1, -1)
  b_up02 = b_up0.reshape(1, -1)
  b_up12 = b_up1.reshape(1, -1)
  b_on2 = b_on.reshape(1, -1)
  b_og2 = b_og.reshape(1, -1)
  b_cf2 = b_cf.reshape(1, -1)
  b_gcu2 = b_gcu.reshape(1, -1)
  b_nphi2 = b_nphi.reshape(1, -1)

  BN = 1000
  BE = 4000
  ngrid = N // BN

  def em_call(We, be2, Wm, bm2):
    return pl.pallas_call(
        _em_body,
        grid=(E // BE,),
        in_specs=[
            pl.BlockSpec((BE, DE), lambda i: (i, 0)),
            _full((DE, DEE)), _full((1, DEE)), _full((DEE + _DH, _DH)),
            _full((1, _DH)),
        ],
        out_specs=pl.BlockSpec((BE, _DH), lambda i: (i, 0)),
        out_shape=jax.ShapeDtypeStruct((E, _DH), f32),
    )(edge_attr, We, be2, Wm, bm2)

  em0 = em_call(W_e0, b_e02, W_msg0, b_msg02)
  em1 = em_call(W_e1, b_e12, W_msg1, b_msg12)

  h0, hm0 = pl.pallas_call(
      _pre_body,
      grid=(ngrid,),
      in_specs=[
          pl.BlockSpec((BN, DF), lambda i: (i, 0)),
          _full((DF, _DH)), _full((1, _DH)), _full((DEE + _DH, _DH)),
      ],
      out_specs=[pl.BlockSpec((BN, _DH), lambda i: (i, 0))] * 2,
      out_shape=[jax.ShapeDtypeStruct((N, _DH), f32)] * 2,
  )(x, W_in, b_in2, W_msg0)

  sc_pass = _sc_message_pass(N, E)
  aggp0 = sc_pass(hm0, em0, src, dst).reshape(_NC, N, _DH)

  h1, hm1 = pl.pallas_call(
      _upd_body,
      grid=(ngrid,),
      in_specs=[
          pl.BlockSpec((BN, _DH), lambda i: (i, 0)),
          pl.BlockSpec((_NC, BN, _DH), lambda i: (0, i, 0)),
          _full((2 * _DH, _DH)), _full((1, _DH)), _full((DEE + _DH, _DH)),
      ],
      out_specs=[pl.BlockSpec((BN, _DH), lambda i: (i, 0))] * 2,
      out_shape=[jax.ShapeDtypeStruct((N, _DH), f32)] * 2,
  )(h0, aggp0, W_up0, b_up02, W_msg1)

  aggp1 = sc_pass(hm1, em1, src, dst).reshape(_NC, N, _DH)

  h2 = pl.pallas_call(
      _upd_last_body,
      grid=(ngrid,),
      in_specs=[
          pl.BlockSpec((BN, _DH), lambda i: (i, 0)),
          pl.BlockSpec((_NC, BN, _DH), lambda i: (0, i, 0)),
          _full((2 * _DH, _DH)), _full((1, _DH)),
      ],
      out_specs=pl.BlockSpec((BN, _DH), lambda i: (i, 0)),
      out_shape=jax.ShapeDtypeStruct((N, _DH), f32),
  )(h1, aggp1, W_up1, b_up12)

  ctx2 = pl.pallas_call(
      _readout_body,
      grid=(ngrid,),
      in_specs=[
          pl.BlockSpec((BN, _DH), lambda i: (i, 0)),
          pl.BlockSpec((BN, 1), lambda i: (i, 0)),
          _full((_DH, _DH)), _full((1, _DH)),
          _full((_DH, _DH)), _full((1, _DH)),
          _full((_DH, _DH)), _full((1, _DH)),
          _full((2 * _DH, 2)),
      ],
      out_specs=pl.BlockSpec((_G, 2), lambda i: (0, 0)),
      out_shape=jax.ShapeDtypeStruct((_G, 2), f32),
      scratch_shapes=[
          pltpu.VMEM((_G, _DH), f32),
          pltpu.VMEM((_G, _DH), f32),
      ],
  )(h2, batch2, W_og, b_og2, W_cf, b_cf2, W_gcu, b_gcu2, W_nphi)

  node_out = pl.pallas_call(
      _fin_body,
      grid=(ngrid,),
      in_specs=[
          pl.BlockSpec((BN, _DH), lambda i: (i, 0)),
          pl.BlockSpec((BN, 1), lambda i: (i, 0)),
          _full((_G, 2)),
          _full((_DH, _DH)), _full((1, _DH)),
          _full((2 * _DH, 2)), _full((1, 2)),
      ],
      out_specs=pl.BlockSpec((BN, 2), lambda i: (i, 0)),
      out_shape=jax.ShapeDtypeStruct((N, 2), f32),
  )(h2, batch2, ctx2, W_on, b_on2, W_nphi, b_nphi2)

  return node_out


# speedup vs baseline: 5.1312x; 1.0146x over previous
import functools

import jax
import jax.numpy as jnp
from jax import lax
from jax.experimental import pallas as pl
from jax.experimental.pallas import tpu as pltpu
from jax.experimental.pallas import tpu_sc as plsc

_NC = 2
_NS = 16
_LANES = 16

_DH = 128
_G = 16


def _sc_message_pass(N, E):
  EPW = E // (_NC * _NS)
  K = 64
  NFULL = EPW // K
  REM = EPW - NFULL * K
  RPB = 8 * (N // (8 * _NS))
  REXTRA = N - RPB * _NS
  assert EPW * _NC * _NS == E and N % 8 == 0
  assert EPW % 8 == 0 and (REM == 0 or REM % 8 == 0)
  assert REXTRA % 8 == 0 and REXTRA < K

  NR = 2
  NE = 3
  NI = 4
  UNROLL = 12
  assert NFULL % UNROLL == 0
  NSUP = NFULL // UNROLL

  mesh = plsc.VectorSubcoreMesh(core_axis_name="c", subcore_axis_name="s")
  scratch = (
      [pltpu.VMEM_SHARED((N, _DH), jnp.float32)]
      + [pltpu.VMEM((K,), jnp.int32)] * (2 * NI)
      + [pltpu.VMEM((K, _DH), jnp.float32)] * (NE + NR)
      + [pltpu.SemaphoreType.DMA] * (NI + NR + 2 * NE)
      + ([pltpu.VMEM((REM,), jnp.int32)] * 2 if REM else [])
  )

  @functools.partial(
      pl.kernel,
      out_type=jax.ShapeDtypeStruct((_NC * N, _DH), jnp.float32),
      mesh=mesh,
      scratch_types=scratch,
  )
  def body(hm, em, srci, dsti, out, agg_sh, *bufs):
    o = 0
    isrc = bufs[o:o + NI]; o += NI
    idst = bufs[o:o + NI]; o += NI
    embuf = bufs[o:o + NE]; o += NE
    rowbuf = bufs[o:o + NR]; o += NR
    isem = bufs[o:o + NI]; o += NI
    gsem = bufs[o:o + NR]; o += NR
    esem = bufs[o:o + NE]; o += NE
    ssem = bufs[o:o + NE]; o += NE
    rem_bufs = bufs[o:]

    cid = lax.axis_index("c")
    sid = lax.axis_index("s")

    zb = rowbuf[0]
    def zrow(r, _):
      for c in range(_DH // _LANES):
        zb[r, pl.ds(c * _LANES, _LANES)] = jnp.zeros((_LANES,), jnp.float32)
      return 0
    lax.fori_loop(0, K, zrow, 0)
    zbase = pl.multiple_of(sid * RPB, 8)
    nz = RPB // K
    rz = RPB - nz * K
    for b in range(nz):
      pltpu.sync_copy(zb, agg_sh.at[pl.ds(zbase + b * K, K)])
    if rz:
      pltpu.sync_copy(zb.at[pl.ds(0, rz)],
                      agg_sh.at[pl.ds(zbase + nz * K, rz)])
    if REXTRA:
      @pl.when(sid == _NS - 1)
      def _():
        pltpu.sync_copy(zb.at[pl.ds(0, REXTRA)],
                        agg_sh.at[pl.ds(N - REXTRA, REXTRA)])
    plsc.subcore_barrier()

    base0 = (cid * _NS + sid) * EPW

    def ebase(c):
      return pl.multiple_of(base0 + c * K, 8)

    def issue_idx(c, i):
      pltpu.async_copy(srci.at[pl.ds(ebase(c), K)], isrc[i], isem[i])
      pltpu.async_copy(dsti.at[pl.ds(ebase(c), K)], idst[i], isem[i])

    def wait_idx(i):
      pltpu.make_async_copy(srci.at[pl.ds(0, K)], isrc[i], isem[i]).wait()
      pltpu.make_async_copy(dsti.at[pl.ds(0, K)], idst[i], isem[i]).wait()

    def issue_data(c, i, r_, e_):
      pltpu.async_copy(hm.at[isrc[i]], rowbuf[r_], gsem[r_])
      pltpu.async_copy(em.at[pl.ds(ebase(c), K)], embuf[e_], esem[e_])

    def wait_data(r_, e_):
      pltpu.make_async_copy(hm.at[pl.ds(0, K)], rowbuf[r_], gsem[r_]).wait()
      pltpu.make_async_copy(em.at[pl.ds(0, K)], embuf[e_], esem[e_]).wait()

    def wait_scatter(e_):
      pltpu.make_async_copy(em.at[pl.ds(0, K)], embuf[e_], ssem[e_]).wait()

    def compute(k, e_, r_):
      emb = embuf[e_]
      rwb = rowbuf[r_]

      @plsc.parallel_loop(0, k, 1, unroll=2)
      def _(r):
        for c in range(_DH // _LANES):
          sl = pl.ds(c * _LANES, _LANES)
          emb[r, sl] = jnp.maximum(rwb[r, sl] + emb[r, sl], 0.0)

    issue_idx(0, 0)
    issue_idx(1, 1)
    wait_idx(0)
    issue_data(0, 0, 0, 0)

    def super_body(it, _):
      for j in range(UNROLL):
        c = it * UNROLL + j
        if j >= 2:
          wait_scatter((j - 2) % NE)
        else:
          @pl.when(it > 0)
          def _(j=j):
            wait_scatter((j - 2) % NE)
        if j + 1 < UNROLL:
          wait_idx((j + 1) % NI)
          issue_data(c + 1, (j + 1) % NI, (j + 1) % NR, (j + 1) % NE)
        else:
          @pl.when(it < NSUP - 1)
          def _(c=c, j=j):
            wait_idx((j + 1) % NI)
            issue_data(c + 1, (j + 1) % NI, (j + 1) % NR, (j + 1) % NE)
        if j + 2 < UNROLL:
          issue_idx(c + 2, (j + 2) % NI)
        else:
          @pl.when(it < NSUP - 1)
          def _(c=c, j=j):
            issue_idx(c + 2, (j + 2) % NI)
        wait_data(j % NR, j % NE)
        compute(K, j % NE, j % NR)
        pltpu.async_copy(embuf[j % NE], agg_sh.at[idst[j % NI]], ssem[j % NE],
                         add=True)
      return 0

    lax.fori_loop(0, NSUP, super_body, 0)
    wait_scatter((NFULL - 2) % NE)
    wait_scatter((NFULL - 1) % NE)

    if REM:
      isr, idr = rem_bufs
      baser = pl.multiple_of(base0 + NFULL * K, 8)
      pltpu.sync_copy(srci.at[pl.ds(baser, REM)], isr)
      pltpu.sync_copy(dsti.at[pl.ds(baser, REM)], idr)
      g = pltpu.async_copy(hm.at[isr], rowbuf[0].at[pl.ds(0, REM)], gsem[0])
      e = pltpu.async_copy(em.at[pl.ds(baser, REM)], embuf[0].at[pl.ds(0, REM)],
                           esem[0])
      g.wait()
      e.wait()
      compute(REM, 0, 0)
      pltpu.sync_copy(embuf[0].at[pl.ds(0, REM)], agg_sh.at[idr], add=True)

    plsc.subcore_barrier()
    obase = pl.multiple_of(cid * N + sid * RPB, 8)
    pltpu.sync_copy(agg_sh.at[pl.ds(zbase, RPB)], out.at[pl.ds(obase, RPB)])
    if REXTRA:
      @pl.when(sid == _NS - 1)
      def _():
        xbase = pl.multiple_of(cid * N + N - REXTRA, 8)
        pltpu.sync_copy(agg_sh.at[pl.ds(N - REXTRA, REXTRA)],
                        out.at[pl.ds(xbase, REXTRA)])

  return body


def _dot(a, b):
  return jnp.dot(a, b, preferred_element_type=jnp.float32)


def _em_body(ea, We, be, Wm, bm, em):
  Wme = Wm[_DH:, :]
  em[...] = _dot(ea[...], _dot(We[...], Wme)) + (_dot(be[...], Wme) + bm[...])


def _pre_body(x, Win, bin_, Wm0, h_out, hm_out):
  h = _dot(x[...], Win[...]) + bin_[...]
  h_out[...] = h
  hm_out[...] = _dot(h, Wm0[: _DH, :])


def _upd_body(h, a, Wu, bu, Wm, ho, hm):
  agg = a[0] + a[1]
  hn = _dot(h[...], Wu[: _DH, :]) + _dot(agg, Wu[_DH:, :]) + bu[...]
  hn = jnp.maximum(hn, 0.0)
  ho[...] = hn
  hm[...] = _dot(hn, Wm[: _DH, :])


def _upd_last_body(h, a, Wu, bu, ho):
  agg = a[0] + a[1]
  hn = _dot(h[...], Wu[: _DH, :]) + _dot(agg, Wu[_DH:, :]) + bu[...]
  ho[...] = jnp.maximum(hn, 0.0)


def _readout_body(h, b, Wog, bog, Wcf, bcf, Wgcu, bgcu, Wnphi, ctx2,
                  pooled, cnt):
  i = pl.program_id(0)

  @pl.when(i == 0)
  def _():
    pooled[...] = jnp.zeros_like(pooled)
    cnt[...] = jnp.zeros_like(cnt)

  bn = h.shape[0]
  oh = (b[...] == lax.broadcasted_iota(jnp.int32, (bn, _G), 1))
  oh = oh.astype(jnp.float32)
  dn = (((0,), (0,)), ((), ()))
  pooled[...] += lax.dot_general(oh, h[...], dn,
                                 preferred_element_type=jnp.float32)
  cnt[...] += lax.dot_general(oh, jnp.ones((bn, _DH), jnp.float32), dn,
                              preferred_element_type=jnp.float32)

  @pl.when(i == pl.num_programs(0) - 1)
  def _():
    pm = pooled[...] / jnp.maximum(cnt[...], 1.0)
    gr = _dot(pm, Wog[...]) + bog[...]
    cx = jnp.maximum(_dot(gr, Wcf[...]) + bcf[...], 0.0)
    cx = _dot(cx, Wgcu[...]) + bgcu[...]
    ctx2[...] = _dot(cx, Wnphi[_DH:, :])


def _fin_body(h, b, ctx2, Won, bon, Wnphi, bnphi, out):
  Wn1 = Wnphi[: _DH, :]
  Wno = _dot(Won[...], Wn1)
  bno = _dot(bon[...], Wn1) + bnphi[...]
  bn = h.shape[0]
  oh = (b[...] == lax.broadcasted_iota(jnp.int32, (bn, _G), 1))
  oh = oh.astype(jnp.float32)
  out[...] = _dot(h[...], Wno) + bno + _dot(oh, ctx2[...])


def _full(shape):
  return pl.BlockSpec(shape, lambda i: tuple(0 for _ in shape))


def kernel(x, edge_index, edge_attr, batch,
           W_in, b_in, W_e0, b_e0, W_e1, b_e1,
           W_msg0, b_msg0, W_up0, b_up0,
           W_msg1, b_msg1, W_up1, b_up1,
           W_on, b_on, W_og, b_og,
           W_cf, b_cf, W_gcu, b_gcu,
           W_nphi, b_nphi):
  N, DF = x.shape
  E = edge_attr.shape[0]
  DE = edge_attr.shape[1]
  DEE = W_e0.shape[1]
  f32 = jnp.float32

  src = edge_index[0]
  dst = edge_index[1]
  batch2 = batch.reshape(N, 1)
  b_in2 = b_in.reshape(1, -1)
  b_e02 = b_e0.reshape(1, -1)
  b_e12 = b_e1.reshape(1, -1)
  b_msg02 = b_msg0.reshape(1, -1)
  b_msg12 = b_msg1.reshape(1, -1)
  b_up02 = b_up0.reshape(1, -1)
  b_up12 = b_up1.reshape(1, -1)
  b_on2 = b_on.reshape(1, -1)
  b_og2 = b_og.reshape(1, -1)
  b_cf2 = b_cf.reshape(1, -1)
  b_gcu2 = b_gcu.reshape(1, -1)
  b_nphi2 = b_nphi.reshape(1, -1)

  BN = 1000
  BE = 4000
  ngrid = N // BN

  def em_call(We, be2, Wm, bm2):
    return pl.pallas_call(
        _em_body,
        grid=(E // BE,),
        in_specs=[
            pl.BlockSpec((BE, DE), lambda i: (i, 0)),
            _full((DE, DEE)), _full((1, DEE)), _full((DEE + _DH, _DH)),
            _full((1, _DH)),
        ],
        out_specs=pl.BlockSpec((BE, _DH), lambda i: (i, 0)),
        out_shape=jax.ShapeDtypeStruct((E, _DH), f32),
    )(edge_attr, We, be2, Wm, bm2)

  em0 = em_call(W_e0, b_e02, W_msg0, b_msg02)
  em1 = em_call(W_e1, b_e12, W_msg1, b_msg12)

  h0, hm0 = pl.pallas_call(
      _pre_body,
      grid=(ngrid,),
      in_specs=[
          pl.BlockSpec((BN, DF), lambda i: (i, 0)),
          _full((DF, _DH)), _full((1, _DH)), _full((DEE + _DH, _DH)),
      ],
      out_specs=[pl.BlockSpec((BN, _DH), lambda i: (i, 0))] * 2,
      out_shape=[jax.ShapeDtypeStruct((N, _DH), f32)] * 2,
  )(x, W_in, b_in2, W_msg0)

  sc_pass = _sc_message_pass(N, E)
  aggp0 = sc_pass(hm0, em0, src, dst).reshape(_NC, N, _DH)

  h1, hm1 = pl.pallas_call(
      _upd_body,
      grid=(ngrid,),
      in_specs=[
          pl.BlockSpec((BN, _DH), lambda i: (i, 0)),
          pl.BlockSpec((_NC, BN, _DH), lambda i: (0, i, 0)),
          _full((2 * _DH, _DH)), _full((1, _DH)), _full((DEE + _DH, _DH)),
      ],
      out_specs=[pl.BlockSpec((BN, _DH), lambda i: (i, 0))] * 2,
      out_shape=[jax.ShapeDtypeStruct((N, _DH), f32)] * 2,
  )(h0, aggp0, W_up0, b_up02, W_msg1)

  aggp1 = sc_pass(hm1, em1, src, dst).reshape(_NC, N, _DH)

  h2 = pl.pallas_call(
      _upd_last_body,
      grid=(ngrid,),
      in_specs=[
          pl.BlockSpec((BN, _DH), lambda i: (i, 0)),
          pl.BlockSpec((_NC, BN, _DH), lambda i: (0, i, 0)),
          _full((2 * _DH, _DH)), _full((1, _DH)),
      ],
      out_specs=pl.BlockSpec((BN, _DH), lambda i: (i, 0)),
      out_shape=jax.ShapeDtypeStruct((N, _DH), f32),
  )(h1, aggp1, W_up1, b_up12)

  ctx2 = pl.pallas_call(
      _readout_body,
      grid=(ngrid,),
      in_specs=[
          pl.BlockSpec((BN, _DH), lambda i: (i, 0)),
          pl.BlockSpec((BN, 1), lambda i: (i, 0)),
          _full((_DH, _DH)), _full((1, _DH)),
          _full((_DH, _DH)), _full((1, _DH)),
          _full((_DH, _DH)), _full((1, _DH)),
          _full((2 * _DH, 2)),
      ],
      out_specs=pl.BlockSpec((_G, 2), lambda i: (0, 0)),
      out_shape=jax.ShapeDtypeStruct((_G, 2), f32),
      scratch_shapes=[
          pltpu.VMEM((_G, _DH), f32),
          pltpu.VMEM((_G, _DH), f32),
      ],
  )(h2, batch2, W_og, b_og2, W_cf, b_cf2, W_gcu, b_gcu2, W_nphi)

  node_out = pl.pallas_call(
      _fin_body,
      grid=(ngrid,),
      in_specs=[
          pl.BlockSpec((BN, _DH), lambda i: (i, 0)),
          pl.BlockSpec((BN, 1), lambda i: (i, 0)),
          _full((_G, 2)),
          _full((_DH, _DH)), _full((1, _DH)),
          _full((2 * _DH, 2)), _full((1, 2)),
      ],
      out_specs=pl.BlockSpec((BN, 2), lambda i: (i, 0)),
      out_shape=jax.ShapeDtypeStruct((N, 2), f32),
  )(h2, batch2, ctx2, W_on, b_on2, W_nphi, b_nphi2)

  return node_out


# speedup vs baseline: 6.0918x; 1.1872x over previous
import functools

import jax
import jax.numpy as jnp
from jax import lax
from jax.experimental import pallas as pl
from jax.experimental.pallas import tpu as pltpu
from jax.experimental.pallas import tpu_sc as plsc

_NC = 2
_NS = 16
_LANES = 16

_DH = 128
_G = 16


def _sc_message_pass(N, E):
  EPW = E // (_NC * _NS)
  K = 64
  NFULL = EPW // K
  REM = EPW - NFULL * K
  RPB = 8 * (N // (8 * _NS))
  REXTRA = N - RPB * _NS
  assert EPW * _NC * _NS == E and N % 8 == 0
  assert EPW % 8 == 0 and K % 8 == 0 and (REM == 0 or REM % 8 == 0)
  assert REXTRA % 8 == 0 and REXTRA < K
  KP = K // 2
  REMP = REM // 2

  NR = 2
  NM = 2
  NI = 4
  UNROLL = 4
  assert NFULL % UNROLL == 0
  NSUP = NFULL // UNROLL

  mesh = plsc.VectorSubcoreMesh(core_axis_name="c", subcore_axis_name="s")
  scratch = (
      [pltpu.VMEM_SHARED((N, _DH), jnp.float32)]
      + [pltpu.VMEM((K,), jnp.int32)] * (2 * NI)
      + [pltpu.VMEM((K, _DH), jnp.float32)] * NR
      + [pltpu.VMEM((KP, _DH), jnp.int32)] * NR
      + [pltpu.VMEM((K, _DH), jnp.float32)] * NM
      + [pltpu.SemaphoreType.DMA] * (NI + 2 * NR + NM)
      + ([pltpu.VMEM((REM,), jnp.int32)] * 2 if REM else [])
  )

  @functools.partial(
      pl.kernel,
      out_type=jax.ShapeDtypeStruct((_NC * N, _DH), jnp.float32),
      mesh=mesh,
      scratch_types=scratch,
  )
  def body(hm, em, srci, dsti, out, agg_sh, *bufs):
    o = 0
    isrc = bufs[o:o + NI]; o += NI
    idst = bufs[o:o + NI]; o += NI
    rowbuf = bufs[o:o + NR]; o += NR
    embuf = bufs[o:o + NR]; o += NR
    msg = bufs[o:o + NM]; o += NM
    isem = bufs[o:o + NI]; o += NI
    gsem = bufs[o:o + NR]; o += NR
    esem = bufs[o:o + NR]; o += NR
    ssem = bufs[o:o + NM]; o += NM
    rem_bufs = bufs[o:]

    cid = lax.axis_index("c")
    sid = lax.axis_index("s")

    zb = msg[0]
    def zrow(r, _):
      for c in range(_DH // _LANES):
        zb[r, pl.ds(c * _LANES, _LANES)] = jnp.zeros((_LANES,), jnp.float32)
      return 0
    lax.fori_loop(0, K, zrow, 0)
    zbase = pl.multiple_of(sid * RPB, 8)
    nz = RPB // K
    rz = RPB - nz * K
    for b in range(nz):
      pltpu.sync_copy(zb, agg_sh.at[pl.ds(zbase + b * K, K)])
    if rz:
      pltpu.sync_copy(zb.at[pl.ds(0, rz)],
                      agg_sh.at[pl.ds(zbase + nz * K, rz)])
    if REXTRA:
      @pl.when(sid == _NS - 1)
      def _():
        pltpu.sync_copy(zb.at[pl.ds(0, REXTRA)],
                        agg_sh.at[pl.ds(N - REXTRA, REXTRA)])
    plsc.subcore_barrier()

    base0 = (cid * _NS + sid) * EPW

    def ebase(c):
      return pl.multiple_of(base0 + c * K, 8)

    def issue_idx(c, i):
      pltpu.async_copy(srci.at[pl.ds(ebase(c), K)], isrc[i], isem[i])
      pltpu.async_copy(dsti.at[pl.ds(ebase(c), K)], idst[i], isem[i])

    def wait_idx(i):
      pltpu.make_async_copy(srci.at[pl.ds(0, K)], isrc[i], isem[i]).wait()
      pltpu.make_async_copy(dsti.at[pl.ds(0, K)], idst[i], isem[i]).wait()

    base0p = base0 // 2

    def epbase(c):
      return pl.multiple_of(base0p + c * KP, 8)

    def issue_data(c, i, r_):
      pltpu.async_copy(hm.at[isrc[i]], rowbuf[r_], gsem[r_])
      pltpu.async_copy(em.at[pl.ds(epbase(c), KP)], embuf[r_], esem[r_])

    def wait_data(r_):
      pltpu.make_async_copy(hm.at[pl.ds(0, K)], rowbuf[r_], gsem[r_]).wait()
      pltpu.make_async_copy(em.at[pl.ds(0, KP)], embuf[r_], esem[r_]).wait()

    def wait_scatter(m_):
      pltpu.make_async_copy(out.at[pl.ds(0, K)], msg[m_], ssem[m_]).wait()

    def compute(k, r_, m_):
      emb = embuf[r_]
      rwb = rowbuf[r_]
      mb = msg[m_]
      himask = jnp.int32(-65536)
      f32 = jnp.float32

      @plsc.parallel_loop(0, k // 2, 1, unroll=2)
      def _(q):
        r0 = q * 2
        for c in range(_DH // _LANES):
          sl = pl.ds(c * _LANES, _LANES)
          eb = emb[q, sl]
          elo = lax.bitcast_convert_type(jnp.left_shift(eb, 16), f32)
          ehi = lax.bitcast_convert_type(eb & himask, f32)
          mb[r0, sl] = jnp.maximum(rwb[r0, sl] + elo, 0.0)
          mb[r0 + 1, sl] = jnp.maximum(rwb[r0 + 1, sl] + ehi, 0.0)

    issue_idx(0, 0)
    issue_idx(1, 1)
    wait_idx(0)
    issue_data(0, 0, 0)

    def super_body(it, _):
      for j in range(UNROLL):
        c = it * UNROLL + j
        if j >= 2:
          wait_scatter((j - 2) % NM)
        else:
          @pl.when(it > 0)
          def _(j=j):
            wait_scatter((j - 2) % NM)
        if j + 1 < UNROLL:
          wait_idx((j + 1) % NI)
          issue_data(c + 1, (j + 1) % NI, (j + 1) % NR)
        else:
          @pl.when(it < NSUP - 1)
          def _(c=c, j=j):
            wait_idx((j + 1) % NI)
            issue_data(c + 1, (j + 1) % NI, (j + 1) % NR)
        if j + 2 < UNROLL:
          issue_idx(c + 2, (j + 2) % NI)
        else:
          @pl.when(it < NSUP - 1)
          def _(c=c, j=j):
            issue_idx(c + 2, (j + 2) % NI)
        wait_data(j % NR)
        compute(K, j % NR, j % NM)
        pltpu.async_copy(msg[j % NM], agg_sh.at[idst[j % NI]], ssem[j % NM],
                         add=True)
      return 0

    lax.fori_loop(0, NSUP, super_body, 0)
    wait_scatter((NFULL - 2) % NM)
    wait_scatter((NFULL - 1) % NM)

    if REM:
      isr, idr = rem_bufs
      baser = pl.multiple_of(base0 + NFULL * K, 8)
      pltpu.sync_copy(srci.at[pl.ds(baser, REM)], isr)
      pltpu.sync_copy(dsti.at[pl.ds(baser, REM)], idr)
      g = pltpu.async_copy(hm.at[isr], rowbuf[0].at[pl.ds(0, REM)], gsem[0])
      e = pltpu.async_copy(em.at[pl.ds(pl.multiple_of(base0p + NFULL * KP, 8),
                                       REMP)],
                           embuf[0].at[pl.ds(0, REMP)], esem[0])
      g.wait()
      e.wait()
      compute(REM, 0, 0)
      pltpu.sync_copy(msg[0].at[pl.ds(0, REM)], agg_sh.at[idr], add=True)

    plsc.subcore_barrier()
    obase = pl.multiple_of(cid * N + sid * RPB, 8)
    pltpu.sync_copy(agg_sh.at[pl.ds(zbase, RPB)], out.at[pl.ds(obase, RPB)])
    if REXTRA:
      @pl.when(sid == _NS - 1)
      def _():
        xbase = pl.multiple_of(cid * N + N - REXTRA, 8)
        pltpu.sync_copy(agg_sh.at[pl.ds(N - REXTRA, REXTRA)],
                        out.at[pl.ds(xbase, REXTRA)])

  return body


def _dot(a, b):
  return jnp.dot(a, b, preferred_element_type=jnp.float32)


def _em_body(ea, We, be, Wm, bm, em):
  Wme = Wm[_DH:, :]
  v = _dot(ea[...], _dot(We[...], Wme)) + (_dot(be[...], Wme) + bm[...])
  em[...] = pltpu.bitcast(v.astype(jnp.bfloat16), jnp.int32)


def _pre_body(x, Win, bin_, Wmh, h_out, hm_out):
  h = _dot(x[...], Win[...]) + bin_[...]
  h_out[...] = h
  hm_out[...] = _dot(h, Wmh[...])


def _upd_body(h, a, Wu, bu, Wmh, ho, hm):
  agg = a[0] + a[1]
  hn = _dot(h[...], Wu[: _DH, :]) + _dot(agg, Wu[_DH:, :]) + bu[...]
  hn = jnp.maximum(hn, 0.0)
  ho[...] = hn
  hm[...] = _dot(hn, Wmh[...])


def _upd_last_body(h, a, Wu, bu, ho):
  agg = a[0] + a[1]
  hn = _dot(h[...], Wu[: _DH, :]) + _dot(agg, Wu[_DH:, :]) + bu[...]
  ho[...] = jnp.maximum(hn, 0.0)


def _readout_body(h, b, Wog, bog, Wcf, bcf, Wgcu, bgcu, Wnphi, ctx2,
                  pooled, cnt):
  i = pl.program_id(0)

  @pl.when(i == 0)
  def _():
    pooled[...] = jnp.zeros_like(pooled)
    cnt[...] = jnp.zeros_like(cnt)

  bn = h.shape[0]
  oh = (b[...] == lax.broadcasted_iota(jnp.int32, (bn, _G), 1))
  oh = oh.astype(jnp.float32)
  dn = (((0,), (0,)), ((), ()))
  pooled[...] += lax.dot_general(oh, h[...], dn,
                                 preferred_element_type=jnp.float32)
  cnt[...] += lax.dot_general(oh, jnp.ones((bn, _DH), jnp.float32), dn,
                              preferred_element_type=jnp.float32)

  @pl.when(i == pl.num_programs(0) - 1)
  def _():
    pm = pooled[...] / jnp.maximum(cnt[...], 1.0)
    gr = _dot(pm, Wog[...]) + bog[...]
    cx = jnp.maximum(_dot(gr, Wcf[...]) + bcf[...], 0.0)
    cx = _dot(cx, Wgcu[...]) + bgcu[...]
    ctx2[...] = _dot(cx, Wnphi[_DH:, :])


def _fin_body(h, b, ctx2, Won, bon, Wnphi, bnphi, out):
  Wn1 = Wnphi[: _DH, :]
  Wno = _dot(Won[...], Wn1)
  bno = _dot(bon[...], Wn1) + bnphi[...]
  bn = h.shape[0]
  oh = (b[...] == lax.broadcasted_iota(jnp.int32, (bn, _G), 1))
  oh = oh.astype(jnp.float32)
  out[...] = _dot(h[...], Wno) + bno + _dot(oh, ctx2[...])


def _full(shape):
  return pl.BlockSpec(shape, lambda i: tuple(0 for _ in shape))


def kernel(x, edge_index, edge_attr, batch,
           W_in, b_in, W_e0, b_e0, W_e1, b_e1,
           W_msg0, b_msg0, W_up0, b_up0,
           W_msg1, b_msg1, W_up1, b_up1,
           W_on, b_on, W_og, b_og,
           W_cf, b_cf, W_gcu, b_gcu,
           W_nphi, b_nphi):
  N, DF = x.shape
  E = edge_attr.shape[0]
  DE = edge_attr.shape[1]
  DEE = W_e0.shape[1]
  f32 = jnp.float32

  src = edge_index[0]
  dst = edge_index[1]
  batch2 = batch.reshape(N, 1)
  b_in2 = b_in.reshape(1, -1)
  b_e02 = b_e0.reshape(1, -1)
  b_e12 = b_e1.reshape(1, -1)
  b_msg02 = b_msg0.reshape(1, -1)
  b_msg12 = b_msg1.reshape(1, -1)
  b_up02 = b_up0.reshape(1, -1)
  b_up12 = b_up1.reshape(1, -1)
  b_on2 = b_on.reshape(1, -1)
  b_og2 = b_og.reshape(1, -1)
  b_cf2 = b_cf.reshape(1, -1)
  b_gcu2 = b_gcu.reshape(1, -1)
  b_nphi2 = b_nphi.reshape(1, -1)

  Wm0h = W_msg0[:_DH]
  Wm1h = W_msg1[:_DH]

  BN = 2000
  BE = 4000
  ngrid = N // BN
  bf16 = jnp.bfloat16

  def em_call(We, be2, Wm, bm2):
    return pl.pallas_call(
        _em_body,
        grid=(E // BE,),
        in_specs=[
            pl.BlockSpec((BE, DE), lambda i: (i, 0)),
            _full((DE, DEE)), _full((1, DEE)), _full((DEE + _DH, _DH)),
            _full((1, _DH)),
        ],
        out_specs=pl.BlockSpec((BE // 2, _DH), lambda i: (i, 0)),
        out_shape=jax.ShapeDtypeStruct((E // 2, _DH), jnp.int32),
    )(edge_attr, We, be2, Wm, bm2)

  em0i = em_call(W_e0, b_e02, W_msg0, b_msg02)
  em1i = em_call(W_e1, b_e12, W_msg1, b_msg12)

  h0, hm0 = pl.pallas_call(
      _pre_body,
      grid=(ngrid,),
      in_specs=[
          pl.BlockSpec((BN, DF), lambda i: (i, 0)),
          _full((DF, _DH)), _full((1, _DH)), _full((_DH, _DH)),
      ],
      out_specs=[pl.BlockSpec((BN, _DH), lambda i: (i, 0))] * 2,
      out_shape=[jax.ShapeDtypeStruct((N, _DH), f32)] * 2,
  )(x, W_in, b_in2, Wm0h)

  sc_pass = _sc_message_pass(N, E)
  aggp0 = sc_pass(hm0, em0i, src, dst).reshape(_NC, N, _DH)

  h1, hm1 = pl.pallas_call(
      _upd_body,
      grid=(ngrid,),
      in_specs=[
          pl.BlockSpec((BN, _DH), lambda i: (i, 0)),
          pl.BlockSpec((_NC, BN, _DH), lambda i: (0, i, 0)),
          _full((2 * _DH, _DH)), _full((1, _DH)), _full((_DH, _DH)),
      ],
      out_specs=[pl.BlockSpec((BN, _DH), lambda i: (i, 0))] * 2,
      out_shape=[jax.ShapeDtypeStruct((N, _DH), f32)] * 2,
  )(h0, aggp0, W_up0, b_up02, Wm1h)

  aggp1 = sc_pass(hm1, em1i, src, dst).reshape(_NC, N, _DH)

  h2 = pl.pallas_call(
      _upd_last_body,
      grid=(ngrid,),
      in_specs=[
          pl.BlockSpec((BN, _DH), lambda i: (i, 0)),
          pl.BlockSpec((_NC, BN, _DH), lambda i: (0, i, 0)),
          _full((2 * _DH, _DH)), _full((1, _DH)),
      ],
      out_specs=pl.BlockSpec((BN, _DH), lambda i: (i, 0)),
      out_shape=jax.ShapeDtypeStruct((N, _DH), f32),
  )(h1, aggp1, W_up1, b_up12)

  ctx2 = pl.pallas_call(
      _readout_body,
      grid=(ngrid,),
      in_specs=[
          pl.BlockSpec((BN, _DH), lambda i: (i, 0)),
          pl.BlockSpec((BN, 1), lambda i: (i, 0)),
          _full((_DH, _DH)), _full((1, _DH)),
          _full((_DH, _DH)), _full((1, _DH)),
          _full((_DH, _DH)), _full((1, _DH)),
          _full((2 * _DH, 2)),
      ],
      out_specs=pl.BlockSpec((_G, 2), lambda i: (0, 0)),
      out_shape=jax.ShapeDtypeStruct((_G, 2), f32),
      scratch_shapes=[
          pltpu.VMEM((_G, _DH), f32),
          pltpu.VMEM((_G, _DH), f32),
      ],
  )(h2, batch2, W_og, b_og2, W_cf, b_cf2, W_gcu, b_gcu2, W_nphi)

  node_out = pl.pallas_call(
      _fin_body,
      grid=(ngrid,),
      in_specs=[
          pl.BlockSpec((BN, _DH), lambda i: (i, 0)),
          pl.BlockSpec((BN, 1), lambda i: (i, 0)),
          _full((_G, 2)),
          _full((_DH, _DH)), _full((1, _DH)),
          _full((2 * _DH, 2)), _full((1, 2)),
      ],
      out_specs=pl.BlockSpec((BN, 2), lambda i: (i, 0)),
      out_shape=jax.ShapeDtypeStruct((N, 2), f32),
  )(h2, batch2, ctx2, W_on, b_on2, W_nphi, b_nphi2)

  return node_out


# speedup vs baseline: 6.1951x; 1.0170x over previous
import functools

import jax
import jax.numpy as jnp
from jax import lax
from jax.experimental import pallas as pl
from jax.experimental.pallas import tpu as pltpu
from jax.experimental.pallas import tpu_sc as plsc

_NC = 2
_NS = 16
_LANES = 16

_DH = 128
_G = 16


def _sc_message_pass(N, E):
  EPW = E // (_NC * _NS)
  K = 64
  NFULL = EPW // K
  REM = EPW - NFULL * K
  RPB = 8 * (N // (8 * _NS))
  REXTRA = N - RPB * _NS
  assert EPW * _NC * _NS == E and N % 8 == 0
  assert EPW % 8 == 0 and K % 8 == 0 and (REM == 0 or REM % 8 == 0)
  assert REXTRA % 8 == 0 and REXTRA < K
  KP = K // 2
  REMP = REM // 2

  NR = 2
  NM = 2
  NI = 4
  UNROLL = 4
  assert NFULL % UNROLL == 0
  NSUP = NFULL // UNROLL

  mesh = plsc.VectorSubcoreMesh(core_axis_name="c", subcore_axis_name="s")
  scratch = (
      [pltpu.VMEM_SHARED((N, _DH), jnp.float32)]
      + [pltpu.VMEM((K,), jnp.int32)] * (2 * NI)
      + [pltpu.VMEM((K, _DH), jnp.float32)] * NR
      + [pltpu.VMEM((KP, _DH), jnp.int32)] * NR
      + [pltpu.VMEM((K, _DH), jnp.float32)] * NM
      + [pltpu.SemaphoreType.DMA] * (NI + 2 * NR + NM)
      + ([pltpu.VMEM((REM,), jnp.int32)] * 2 if REM else [])
  )

  @functools.partial(
      pl.kernel,
      out_type=jax.ShapeDtypeStruct((_NC * N, _DH), jnp.float32),
      mesh=mesh,
      scratch_types=scratch,
  )
  def body(hm, em, srci, dsti, out, agg_sh, *bufs):
    o = 0
    isrc = bufs[o:o + NI]; o += NI
    idst = bufs[o:o + NI]; o += NI
    rowbuf = bufs[o:o + NR]; o += NR
    embuf = bufs[o:o + NR]; o += NR
    msg = bufs[o:o + NM]; o += NM
    isem = bufs[o:o + NI]; o += NI
    gsem = bufs[o:o + NR]; o += NR
    esem = bufs[o:o + NR]; o += NR
    ssem = bufs[o:o + NM]; o += NM
    rem_bufs = bufs[o:]

    cid = lax.axis_index("c")
    sid = lax.axis_index("s")

    zb = msg[0]
    def zrow(r, _):
      for c in range(_DH // _LANES):
        zb[r, pl.ds(c * _LANES, _LANES)] = jnp.zeros((_LANES,), jnp.float32)
      return 0
    lax.fori_loop(0, K, zrow, 0)
    zbase = pl.multiple_of(sid * RPB, 8)
    nz = RPB // K
    rz = RPB - nz * K
    for b in range(nz):
      pltpu.sync_copy(zb, agg_sh.at[pl.ds(zbase + b * K, K)])
    if rz:
      pltpu.sync_copy(zb.at[pl.ds(0, rz)],
                      agg_sh.at[pl.ds(zbase + nz * K, rz)])
    if REXTRA:
      @pl.when(sid == _NS - 1)
      def _():
        pltpu.sync_copy(zb.at[pl.ds(0, REXTRA)],
                        agg_sh.at[pl.ds(N - REXTRA, REXTRA)])
    plsc.subcore_barrier()

    base0 = (cid * _NS + sid) * EPW

    def ebase(c):
      return pl.multiple_of(base0 + c * K, 8)

    def issue_idx(c, i):
      pltpu.async_copy(srci.at[pl.ds(ebase(c), K)], isrc[i], isem[i])
      pltpu.async_copy(dsti.at[pl.ds(ebase(c), K)], idst[i], isem[i])

    def wait_idx(i):
      pltpu.make_async_copy(srci.at[pl.ds(0, K)], isrc[i], isem[i]).wait()
      pltpu.make_async_copy(dsti.at[pl.ds(0, K)], idst[i], isem[i]).wait()

    base0p = base0 // 2

    def epbase(c):
      return pl.multiple_of(base0p + c * KP, 8)

    def issue_data(c, i, r_):
      pltpu.async_copy(hm.at[isrc[i]], rowbuf[r_], gsem[r_])
      pltpu.async_copy(em.at[pl.ds(epbase(c), KP)], embuf[r_], esem[r_])

    def wait_data(r_):
      pltpu.make_async_copy(hm.at[pl.ds(0, K)], rowbuf[r_], gsem[r_]).wait()
      pltpu.make_async_copy(em.at[pl.ds(0, KP)], embuf[r_], esem[r_]).wait()

    def wait_scatter(m_):
      pltpu.make_async_copy(out.at[pl.ds(0, K)], msg[m_], ssem[m_]).wait()

    def compute(k, r_, m_):
      emb = embuf[r_]
      rwb = rowbuf[r_]
      mb = msg[m_]
      himask = jnp.int32(-65536)
      f32 = jnp.float32

      @plsc.parallel_loop(0, k // 2, 1, unroll=2)
      def _(q):
        r0 = q * 2
        for c in range(_DH // _LANES):
          sl = pl.ds(c * _LANES, _LANES)
          eb = emb[q, sl]
          elo = lax.bitcast_convert_type(jnp.left_shift(eb, 16), f32)
          ehi = lax.bitcast_convert_type(eb & himask, f32)
          mb[r0, sl] = jnp.maximum(rwb[r0, sl] + elo, 0.0)
          mb[r0 + 1, sl] = jnp.maximum(rwb[r0 + 1, sl] + ehi, 0.0)

    issue_idx(0, 0)
    issue_idx(1, 1)
    wait_idx(0)
    issue_data(0, 0, 0)

    def super_body(it, _):
      for j in range(UNROLL):
        c = it * UNROLL + j
        if j >= 2:
          wait_scatter((j - 2) % NM)
        else:
          @pl.when(it > 0)
          def _(j=j):
            wait_scatter((j - 2) % NM)
        if j + 1 < UNROLL:
          wait_idx((j + 1) % NI)
          issue_data(c + 1, (j + 1) % NI, (j + 1) % NR)
        else:
          @pl.when(it < NSUP - 1)
          def _(c=c, j=j):
            wait_idx((j + 1) % NI)
            issue_data(c + 1, (j + 1) % NI, (j + 1) % NR)
        if j + 2 < UNROLL:
          issue_idx(c + 2, (j + 2) % NI)
        else:
          @pl.when(it < NSUP - 1)
          def _(c=c, j=j):
            issue_idx(c + 2, (j + 2) % NI)
        wait_data(j % NR)
        compute(K, j % NR, j % NM)
        pltpu.async_copy(msg[j % NM], agg_sh.at[idst[j % NI]], ssem[j % NM],
                         add=True)
      return 0

    lax.fori_loop(0, NSUP, super_body, 0)
    wait_scatter((NFULL - 2) % NM)
    wait_scatter((NFULL - 1) % NM)

    if REM:
      isr, idr = rem_bufs
      baser = pl.multiple_of(base0 + NFULL * K, 8)
      pltpu.sync_copy(srci.at[pl.ds(baser, REM)], isr)
      pltpu.sync_copy(dsti.at[pl.ds(baser, REM)], idr)
      g = pltpu.async_copy(hm.at[isr], rowbuf[0].at[pl.ds(0, REM)], gsem[0])
      e = pltpu.async_copy(em.at[pl.ds(pl.multiple_of(base0p + NFULL * KP, 8),
                                       REMP)],
                           embuf[0].at[pl.ds(0, REMP)], esem[0])
      g.wait()
      e.wait()
      compute(REM, 0, 0)
      pltpu.sync_copy(msg[0].at[pl.ds(0, REM)], agg_sh.at[idr], add=True)

    plsc.subcore_barrier()
    obase = pl.multiple_of(cid * N + sid * RPB, 8)
    pltpu.sync_copy(agg_sh.at[pl.ds(zbase, RPB)], out.at[pl.ds(obase, RPB)])
    if REXTRA:
      @pl.when(sid == _NS - 1)
      def _():
        xbase = pl.multiple_of(cid * N + N - REXTRA, 8)
        pltpu.sync_copy(agg_sh.at[pl.ds(N - REXTRA, REXTRA)],
                        out.at[pl.ds(xbase, REXTRA)])

  return body


def _dot(a, b):
  return jnp.dot(a, b, preferred_element_type=jnp.float32)


def _em_body(ea, We, be, Wm, bm, em):
  Wme = Wm[_DH:, :]
  v = _dot(ea[...], _dot(We[...], Wme)) + (_dot(be[...], Wme) + bm[...])
  em[...] = pltpu.bitcast(v.astype(jnp.bfloat16), jnp.int32)


def _pre_body(x, Win, bin_, Wmh, h_out, hm_out):
  h = _dot(x[...], Win[...]) + bin_[...]
  h_out[...] = h
  hm_out[...] = _dot(h, Wmh[...])


def _upd_body(h, a, Wu, bu, Wmh, ho, hm):
  agg = a[0] + a[1]
  hn = _dot(h[...], Wu[: _DH, :]) + _dot(agg, Wu[_DH:, :]) + bu[...]
  hn = jnp.maximum(hn, 0.0)
  ho[...] = hn
  hm[...] = _dot(hn, Wmh[...])


def _tail_body(h, a, Wu, bu, b2, Wog, bog, Wcf, bcf, Wgcu, bgcu,
               Won, bon, Wnphi, bnphi, out):
  agg = a[0] + a[1]
  h2 = jnp.maximum(
      _dot(h[...], Wu[: _DH, :]) + _dot(agg, Wu[_DH:, :]) + bu[...], 0.0)
  n = h2.shape[0]
  oh = (b2[...] == lax.broadcasted_iota(jnp.int32, (n, _G), 1))
  oh = oh.astype(jnp.float32)
  dn = (((0,), (0,)), ((), ()))
  pooled = lax.dot_general(oh, h2, dn, preferred_element_type=jnp.float32)
  cnt = lax.dot_general(oh, jnp.ones((n, _DH), jnp.float32), dn,
                        preferred_element_type=jnp.float32)
  pm = pooled / jnp.maximum(cnt, 1.0)
  gr = _dot(pm, Wog[...]) + bog[...]
  cx = jnp.maximum(_dot(gr, Wcf[...]) + bcf[...], 0.0)
  cx = _dot(cx, Wgcu[...]) + bgcu[...]
  ctx2 = _dot(cx, Wnphi[_DH:, :])
  Wn1 = Wnphi[: _DH, :]
  Wno = _dot(Won[...], Wn1)
  bno = _dot(bon[...], Wn1) + bnphi[...]
  out[...] = _dot(h2, Wno) + bno + _dot(oh, ctx2)


def _readout_body(h, b, Wog, bog, Wcf, bcf, Wgcu, bgcu, Wnphi, ctx2,
                  pooled, cnt):
  i = pl.program_id(0)

  @pl.when(i == 0)
  def _():
    pooled[...] = jnp.zeros_like(pooled)
    cnt[...] = jnp.zeros_like(cnt)

  bn = h.shape[0]
  oh = (b[...] == lax.broadcasted_iota(jnp.int32, (bn, _G), 1))
  oh = oh.astype(jnp.float32)
  dn = (((0,), (0,)), ((), ()))
  pooled[...] += lax.dot_general(oh, h[...], dn,
                                 preferred_element_type=jnp.float32)
  cnt[...] += lax.dot_general(oh, jnp.ones((bn, _DH), jnp.float32), dn,
                              preferred_element_type=jnp.float32)

  @pl.when(i == pl.num_programs(0) - 1)
  def _():
    pm = pooled[...] / jnp.maximum(cnt[...], 1.0)
    gr = _dot(pm, Wog[...]) + bog[...]
    cx = jnp.maximum(_dot(gr, Wcf[...]) + bcf[...], 0.0)
    cx = _dot(cx, Wgcu[...]) + bgcu[...]
    ctx2[...] = _dot(cx, Wnphi[_DH:, :])


def _fin_body(h, b, ctx2, Won, bon, Wnphi, bnphi, out):
  Wn1 = Wnphi[: _DH, :]
  Wno = _dot(Won[...], Wn1)
  bno = _dot(bon[...], Wn1) + bnphi[...]
  bn = h.shape[0]
  oh = (b[...] == lax.broadcasted_iota(jnp.int32, (bn, _G), 1))
  oh = oh.astype(jnp.float32)
  out[...] = _dot(h[...], Wno) + bno + _dot(oh, ctx2[...])


def _full(shape):
  return pl.BlockSpec(shape, lambda i: tuple(0 for _ in shape))


def kernel(x, edge_index, edge_attr, batch,
           W_in, b_in, W_e0, b_e0, W_e1, b_e1,
           W_msg0, b_msg0, W_up0, b_up0,
           W_msg1, b_msg1, W_up1, b_up1,
           W_on, b_on, W_og, b_og,
           W_cf, b_cf, W_gcu, b_gcu,
           W_nphi, b_nphi):
  N, DF = x.shape
  E = edge_attr.shape[0]
  DE = edge_attr.shape[1]
  DEE = W_e0.shape[1]
  f32 = jnp.float32

  src = edge_index[0]
  dst = edge_index[1]
  batch2 = batch.reshape(N, 1)
  b_in2 = b_in.reshape(1, -1)
  b_e02 = b_e0.reshape(1, -1)
  b_e12 = b_e1.reshape(1, -1)
  b_msg02 = b_msg0.reshape(1, -1)
  b_msg12 = b_msg1.reshape(1, -1)
  b_up02 = b_up0.reshape(1, -1)
  b_up12 = b_up1.reshape(1, -1)
  b_on2 = b_on.reshape(1, -1)
  b_og2 = b_og.reshape(1, -1)
  b_cf2 = b_cf.reshape(1, -1)
  b_gcu2 = b_gcu.reshape(1, -1)
  b_nphi2 = b_nphi.reshape(1, -1)

  Wm0h = W_msg0[:_DH]
  Wm1h = W_msg1[:_DH]

  BN = 2000
  BE = 4000
  ngrid = N // BN
  bf16 = jnp.bfloat16

  def em_call(We, be2, Wm, bm2):
    return pl.pallas_call(
        _em_body,
        grid=(E // BE,),
        in_specs=[
            pl.BlockSpec((BE, DE), lambda i: (i, 0)),
            _full((DE, DEE)), _full((1, DEE)), _full((DEE + _DH, _DH)),
            _full((1, _DH)),
        ],
        out_specs=pl.BlockSpec((BE // 2, _DH), lambda i: (i, 0)),
        out_shape=jax.ShapeDtypeStruct((E // 2, _DH), jnp.int32),
    )(edge_attr, We, be2, Wm, bm2)

  em0i = em_call(W_e0, b_e02, W_msg0, b_msg02)
  em1i = em_call(W_e1, b_e12, W_msg1, b_msg12)

  h0, hm0 = pl.pallas_call(
      _pre_body,
      grid=(ngrid,),
      in_specs=[
          pl.BlockSpec((BN, DF), lambda i: (i, 0)),
          _full((DF, _DH)), _full((1, _DH)), _full((_DH, _DH)),
      ],
      out_specs=[pl.BlockSpec((BN, _DH), lambda i: (i, 0))] * 2,
      out_shape=[jax.ShapeDtypeStruct((N, _DH), f32)] * 2,
  )(x, W_in, b_in2, Wm0h)

  sc_pass = _sc_message_pass(N, E)
  aggp0 = sc_pass(hm0, em0i, src, dst).reshape(_NC, N, _DH)

  h1, hm1 = pl.pallas_call(
      _upd_body,
      grid=(ngrid,),
      in_specs=[
          pl.BlockSpec((BN, _DH), lambda i: (i, 0)),
          pl.BlockSpec((_NC, BN, _DH), lambda i: (0, i, 0)),
          _full((2 * _DH, _DH)), _full((1, _DH)), _full((_DH, _DH)),
      ],
      out_specs=[pl.BlockSpec((BN, _DH), lambda i: (i, 0))] * 2,
      out_shape=[jax.ShapeDtypeStruct((N, _DH), f32)] * 2,
  )(h0, aggp0, W_up0, b_up02, Wm1h)

  aggp1 = sc_pass(hm1, em1i, src, dst).reshape(_NC, N, _DH)

  node_out = pl.pallas_call(
      _tail_body,
      out_shape=jax.ShapeDtypeStruct((N, 2), f32),
  )(h1, aggp1, W_up1, b_up12, batch2, W_og, b_og2, W_cf, b_cf2,
    W_gcu, b_gcu2, W_on, b_on2, W_nphi, b_nphi2)

  return node_out
